# baseline (speedup 1.0000x reference)
import functools
import math

import jax
import jax.numpy as jnp
from jax.experimental import pallas as pl
from jax.experimental.pallas import tpu as pltpu

S = 2048
C = 768
HD = 64
H = C // HD
EPS = float(jnp.finfo(jnp.float32).eps)

BS = 512
BQ = 256
BH = 12


NT = (((1,), (1,)), ((), ()))


def _qkv_kernel(x_ref, wq_ref, wk_ref, wv_ref, b_ref,
                r2_ref, g_ref, e_ref,
                q_ref, k_ref, v_ref):
    xb = x_ref[:]
    e = e_ref[:]

    q = jax.lax.dot_general(xb, wq_ref[:], NT,
                            preferred_element_type=jnp.float32) + b_ref[0:1, :]
    k = jax.lax.dot_general(xb, wk_ref[:], NT,
                            preferred_element_type=jnp.float32) + b_ref[1:2, :]
    v = jax.lax.dot_general(xb, wv_ref[:], NT,
                            preferred_element_type=jnp.float32) + b_ref[2:3, :]

    def split_dot(t, dims):
        hi = t.astype(jnp.bfloat16).astype(jnp.float32)
        lo = t - hi
        f = lambda a: jax.lax.dot_general(a, e, dims,
                                          preferred_element_type=jnp.float32)
        return f(hi) + f(lo)

    def headnorm(t):
        ss = split_dot(t * t, (((1,), (0,)), ((), ())))
        r = jax.lax.rsqrt(ss * (1.0 / HD) + EPS)
        rb = split_dot(r, (((1,), (1,)), ((), ())))
        return t * rb

    r2 = r2_ref[:]
    even64 = jax.lax.broadcasted_iota(jnp.int32, (BS, HD), 1) % 2 == 0
    ce = jnp.where(even64, r2, pltpu.roll(r2, 1, 1))
    se = jnp.where(even64, pltpu.roll(r2, HD - 1, 1), r2)
    ceq = ce * g_ref[0:1, :]
    seq = se * g_ref[1:2, :]
    cek = ce * g_ref[2:3, :]
    sek = se * g_ref[3:4, :]

    lane = jax.lax.broadcasted_iota(jnp.int32, (BS, C), 1)
    even = (lane % 2) == 0

    def partner(t):
        left = pltpu.roll(t, C - 1, 1)
        right = pltpu.roll(t, 1, 1)
        return jnp.where(even, -left, right)

    qn = headnorm(q)
    kn = headnorm(k)
    qp = partner(qn)
    kp = partner(kn)

    v16 = v.astype(jnp.bfloat16)
    dcol = (jax.lax.broadcasted_iota(jnp.int32, (BS, HD), 1) == 0
            ).astype(jnp.bfloat16)
    for h in range(H):
        sl = slice(h * HD, (h + 1) * HD)
        q_ref[h] = qn[:, sl] * ceq + qp[:, sl] * seq
        k_ref[h] = kn[:, sl] * cek + kp[:, sl] * sek
        v_ref[h] = jnp.concatenate([v16[:, sl], dcol], axis=1)


def _attn_kernel(q_ref, k_ref, v_ref, wo_ref, bo_ref, o_ref, acc_ref):
    h2 = pl.program_id(1)

    def one_head(j):
        qb = q_ref[j]
        kb = k_ref[j]
        vb = v_ref[j]
        s = jax.lax.dot_general(qb, kb, (((1,), (1,)), ((), ())),
                                preferred_element_type=jnp.float32)
        m = jnp.max(s, axis=1, keepdims=True)
        p16 = jnp.exp((s - m) * (1.0 / math.sqrt(HD))).astype(jnp.bfloat16)
        oa = jnp.dot(p16, vb, preferred_element_type=jnp.float32)
        denom = oa[:, HD:HD + 1]
        return oa[:, :HD] * (1.0 / denom)

    acc_ref[h2] = jnp.concatenate([one_head(j) for j in range(BH)], axis=1)

    @pl.when(h2 == H // BH - 1)
    def _():
        full = jnp.concatenate([acc_ref[t] for t in range(H // BH)], axis=1)
        o_ref[:] = jax.lax.dot_general(full, wo_ref[:], NT,
                                       preferred_element_type=jnp.float32) + bo_ref[:]


@jax.jit
def kernel(x, rope, Wq, bq, Wk, bk, Wv, bv, gq, gk, Wo, bo):
    f32 = jnp.float32
    r2d = rope.reshape(S, HD)

    def pairswap(v):
        v2 = v.reshape(-1, 2)
        return jnp.stack([v2[:, 1], v2[:, 0]], axis=-1).reshape(-1)

    garr = jnp.stack([gq, pairswap(gq), gk, pairswap(gk)])
    b_all = jnp.stack([bq, bk, bv])
    eye = jnp.repeat(jnp.eye(H, dtype=f32), HD, axis=0)

    row_spec = pl.BlockSpec((BS, C), lambda i: (i, 0))
    full_spec = pl.BlockSpec((C, C), lambda i: (0, 0))
    qkv_out_spec = pl.BlockSpec((H, BS, HD), lambda i: (0, i, 0))

    q3, k3, v3 = pl.pallas_call(
        _qkv_kernel,
        grid=(S // BS,),
        in_specs=[
            row_spec,
            full_spec, full_spec, full_spec,
            pl.BlockSpec((3, C), lambda i: (0, 0)),
            pl.BlockSpec((BS, HD), lambda i: (i, 0)),
            pl.BlockSpec((4, HD), lambda i: (0, 0)),
            pl.BlockSpec((C, H), lambda i: (0, 0)),
        ],
        out_specs=[qkv_out_spec, qkv_out_spec,
                   pl.BlockSpec((H, BS, 2 * HD), lambda i: (0, i, 0))],
        out_shape=[jax.ShapeDtypeStruct((H, S, HD), f32)] * 2
        + [jax.ShapeDtypeStruct((H, S, 2 * HD), jnp.bfloat16)],
    )(x, Wq, Wk, Wv, b_all, r2d, garr, eye)

    out = pl.pallas_call(
        _attn_kernel,
        grid=(S // BQ, H // BH),
        in_specs=[
            pl.BlockSpec((BH, BQ, HD), lambda i, h: (h, i, 0)),
            pl.BlockSpec((BH, S, HD), lambda i, h: (h, 0, 0)),
            pl.BlockSpec((BH, S, 2 * HD), lambda i, h: (h, 0, 0)),
            pl.BlockSpec((C, C), lambda i, h: (0, 0)),
            pl.BlockSpec((1, C), lambda i, h: (0, 0)),
        ],
        out_specs=pl.BlockSpec((BQ, C), lambda i, h: (i, 0)),
        out_shape=jax.ShapeDtypeStruct((S, C), f32),
        scratch_shapes=[pltpu.VMEM((H // BH, BQ, BH * HD), jnp.float32)],
    )(q3, k3, v3, Wo, bo[None, :])

    return out


# speedup vs baseline: 1.1393x; 1.1393x over previous
import functools
import math

import jax
import jax.numpy as jnp
from jax.experimental import pallas as pl
from jax.experimental.pallas import tpu as pltpu

S = 2048
C = 768
HD = 64
H = C // HD
EPS = float(jnp.finfo(jnp.float32).eps)

BS = 512
BQ = 512
BH = 6


NT = (((1,), (1,)), ((), ()))


def _qkv_kernel(x_ref, wq_ref, wk_ref, wv_ref, b_ref,
                r2_ref, g_ref, e_ref,
                q_ref, k_ref, v_ref):
    xb = x_ref[:]
    e = e_ref[:]

    q = jax.lax.dot_general(xb, wq_ref[:], NT,
                            preferred_element_type=jnp.float32) + b_ref[0:1, :]
    k = jax.lax.dot_general(xb, wk_ref[:], NT,
                            preferred_element_type=jnp.float32) + b_ref[1:2, :]
    v = jax.lax.dot_general(xb, wv_ref[:], NT,
                            preferred_element_type=jnp.float32) + b_ref[2:3, :]

    def split_dot(t, dims):
        hi = t.astype(jnp.bfloat16).astype(jnp.float32)
        lo = t - hi
        f = lambda a: jax.lax.dot_general(a, e, dims,
                                          preferred_element_type=jnp.float32)
        return f(hi) + f(lo)

    def headnorm(t):
        ss = split_dot(t * t, (((1,), (0,)), ((), ())))
        r = jax.lax.rsqrt(ss * (1.0 / HD) + EPS)
        rb = split_dot(r, (((1,), (1,)), ((), ())))
        return t * rb

    r2 = r2_ref[:]
    even64 = jax.lax.broadcasted_iota(jnp.int32, (BS, HD), 1) % 2 == 0
    ce = jnp.where(even64, r2, pltpu.roll(r2, 1, 1))
    se = jnp.where(even64, pltpu.roll(r2, HD - 1, 1), r2)
    ceq = ce * g_ref[0:1, :] * (1.0 / math.sqrt(HD))
    seq = se * g_ref[1:2, :] * (1.0 / math.sqrt(HD))
    cek = ce * g_ref[2:3, :]
    sek = se * g_ref[3:4, :]

    lane = jax.lax.broadcasted_iota(jnp.int32, (BS, C), 1)
    even = (lane % 2) == 0

    def partner(t):
        left = pltpu.roll(t, C - 1, 1)
        right = pltpu.roll(t, 1, 1)
        return jnp.where(even, -left, right)

    qn = headnorm(q)
    kn = headnorm(k)
    qp = partner(qn)
    kp = partner(kn)

    v16 = v.astype(jnp.bfloat16)
    dcol = (jax.lax.broadcasted_iota(jnp.int32, (BS, HD), 1) == 0
            ).astype(jnp.bfloat16)
    for h in range(H):
        sl = slice(h * HD, (h + 1) * HD)
        q_ref[h] = qn[:, sl] * ceq + qp[:, sl] * seq
        k_ref[h] = kn[:, sl] * cek + kp[:, sl] * sek
        v_ref[h] = jnp.concatenate([v16[:, sl], dcol], axis=1)


def _attn_kernel(q_ref, k_ref, v_ref, wo_ref, bo_ref, o_ref, acc_ref):
    h2 = pl.program_id(1)

    def one_head(j):
        qb = q_ref[j]
        kb = k_ref[j]
        vb = v_ref[j]
        s = jax.lax.dot_general(qb, kb, (((1,), (1,)), ((), ())),
                                preferred_element_type=jnp.float32)
        m = jnp.max(s, axis=1, keepdims=True)
        p16 = jnp.exp(s - m).astype(jnp.bfloat16)
        oa = jnp.dot(p16, vb, preferred_element_type=jnp.float32)
        denom = oa[:, HD:HD + 1]
        return oa[:, :HD] * (1.0 / denom)

    acc_ref[h2] = jnp.concatenate([one_head(j) for j in range(BH)], axis=1)

    @pl.when(h2 == H // BH - 1)
    def _():
        full = jnp.concatenate([acc_ref[t] for t in range(H // BH)], axis=1)
        o_ref[:] = jax.lax.dot_general(full, wo_ref[:], NT,
                                       preferred_element_type=jnp.float32) + bo_ref[:]


@jax.jit
def kernel(x, rope, Wq, bq, Wk, bk, Wv, bv, gq, gk, Wo, bo):
    f32 = jnp.float32
    r2d = rope.reshape(S, HD)

    def pairswap(v):
        v2 = v.reshape(-1, 2)
        return jnp.stack([v2[:, 1], v2[:, 0]], axis=-1).reshape(-1)

    garr = jnp.stack([gq, pairswap(gq), gk, pairswap(gk)])
    b_all = jnp.stack([bq, bk, bv])
    eye = jnp.repeat(jnp.eye(H, dtype=f32), HD, axis=0)

    row_spec = pl.BlockSpec((BS, C), lambda i: (i, 0))
    full_spec = pl.BlockSpec((C, C), lambda i: (0, 0))
    qkv_out_spec = pl.BlockSpec((H, BS, HD), lambda i: (0, i, 0))

    q3, k3, v3 = pl.pallas_call(
        _qkv_kernel,
        grid=(S // BS,),
        in_specs=[
            row_spec,
            full_spec, full_spec, full_spec,
            pl.BlockSpec((3, C), lambda i: (0, 0)),
            pl.BlockSpec((BS, HD), lambda i: (i, 0)),
            pl.BlockSpec((4, HD), lambda i: (0, 0)),
            pl.BlockSpec((C, H), lambda i: (0, 0)),
        ],
        out_specs=[qkv_out_spec, qkv_out_spec,
                   pl.BlockSpec((H, BS, 2 * HD), lambda i: (0, i, 0))],
        out_shape=[jax.ShapeDtypeStruct((H, S, HD), f32)] * 2
        + [jax.ShapeDtypeStruct((H, S, 2 * HD), jnp.bfloat16)],
    )(x, Wq, Wk, Wv, b_all, r2d, garr, eye)

    out = pl.pallas_call(
        _attn_kernel,
        grid=(S // BQ, H // BH),
        in_specs=[
            pl.BlockSpec((BH, BQ, HD), lambda i, h: (h, i, 0)),
            pl.BlockSpec((BH, S, HD), lambda i, h: (h, 0, 0)),
            pl.BlockSpec((BH, S, 2 * HD), lambda i, h: (h, 0, 0)),
            pl.BlockSpec((C, C), lambda i, h: (0, 0)),
            pl.BlockSpec((1, C), lambda i, h: (0, 0)),
        ],
        out_specs=pl.BlockSpec((BQ, C), lambda i, h: (i, 0)),
        out_shape=jax.ShapeDtypeStruct((S, C), f32),
        scratch_shapes=[pltpu.VMEM((H // BH, BQ, BH * HD), jnp.float32)],
    )(q3, k3, v3, Wo, bo[None, :])

    return out
